# baseline (device time: 149792 ns/iter reference)
import jax
import jax.numpy as jnp
from jax import lax
from jax.experimental import pallas as pl
from jax.experimental.pallas import tpu as pltpu

N_DEV = 8

_sem_signal = getattr(pl, "semaphore_signal", None) or pltpu.semaphore_signal
_sem_wait = getattr(pl, "semaphore_wait", None) or pltpu.semaphore_wait
_DID = getattr(pl, "DeviceIdType", None) or pltpu.DeviceIdType
_CompilerParams = getattr(pltpu, "CompilerParams", None) or pltpu.TPUCompilerParams


def kernel(x, Wq, K_ext, V_ext, Wo):
    B, SQ, DM = x.shape
    _, HD = Wq.shape
    _, SKV, H, DH = K_ext.shape
    HPER = HD // DH
    ROWS = B * SQ
    PACKW = 2 * HD

    xb = x.astype(jnp.bfloat16).reshape(ROWS, DM)
    wqb = Wq.astype(jnp.bfloat16) * jnp.bfloat16(0.125)
    wot = Wo.astype(jnp.bfloat16).T
    pk = jnp.concatenate([wqb, wot], axis=1)
    kt = jnp.transpose(K_ext, (0, 2, 3, 1)).astype(jnp.bfloat16)
    vt = jnp.transpose(V_ext, (0, 2, 3, 1)).astype(jnp.bfloat16)

    def body(x_ref, pk_ref, k_ref, v_ref, out_ref,
             packs, q_ref, ctx_ref, acc, ssems, rsems):
        my = lax.axis_index("i")
        right = lax.rem(my + 1, N_DEV)
        left = lax.rem(my + N_DEV - 1, N_DEV)
        anti = lax.rem(my + 4, N_DEV)

        barrier_sem = pltpu.get_barrier_semaphore()
        for nbr in (left, right, anti):
            _sem_signal(barrier_sem, inc=1, device_id=(nbr,),
                        device_id_type=_DID.MESH)
        _sem_wait(barrier_sem, 3)

        packs[0] = pk_ref[...]
        acc[...] = jnp.zeros_like(acc)

        r4 = (lax.broadcasted_iota(jnp.int32, (SQ, SKV), 0) // 64) % 4
        c4 = (lax.broadcasted_iota(jnp.int32, (SQ, SKV), 1) // 64) % 4
        bias = jnp.where(r4 == c4, 0.0, -1e9).astype(jnp.float32)

        SENDS = (
            (0, 1, right, 0),
            (0, 2, left, 1),
            (0, 3, anti, 2),
            (1, 4, right, 3),
            (2, 5, left, 4),
            (4, 6, right, 5),
            (5, 7, left, 6),
        )

        def rdma(k):
            src, dst, dev, si = SENDS[k]
            return pltpu.make_async_remote_copy(
                src_ref=packs.at[src], dst_ref=packs.at[dst],
                send_sem=ssems.at[si], recv_sem=rsems.at[dst],
                device_id=(dev,), device_id_type=_DID.MESH)

        def wait_recv(slot):
            r = pltpu.make_async_remote_copy(
                src_ref=packs.at[slot], dst_ref=packs.at[slot],
                send_sem=ssems.at[0], recv_sem=rsems.at[slot],
                device_id=(right,), device_id_type=_DID.MESH)
            r.wait_recv()

        def attend(t, j):
            q_ref[...] = jnp.dot(
                x_ref[...], packs[t, :, :HD],
                preferred_element_type=jnp.float32).astype(jnp.bfloat16)

            def head_pair(hl2, carry):
                base = hl2 * (2 * DH)
                for b in range(B):
                    q2 = q_ref[b * SQ:(b + 1) * SQ, pl.ds(base, 2 * DH)]
                    cs = []
                    for sub in range(2):
                        khead = j * HPER + hl2 * 2 + sub
                        qbh = q2[:, sub * DH:(sub + 1) * DH]
                        kbh = k_ref[b, khead]
                        s = lax.dot_general(
                            qbh, kbh, (((1,), (0,)), ((), ())),
                            preferred_element_type=jnp.float32)
                        e = jnp.exp(s + bias)
                        den = jnp.sum(e, axis=1, keepdims=True)
                        c = lax.dot_general(
                            e.astype(jnp.bfloat16), v_ref[b, khead],
                            (((1,), (1,)), ((), ())),
                            preferred_element_type=jnp.float32)
                        cs.append((c * (1.0 / den)).astype(jnp.bfloat16))
                    ctx_ref[b * SQ:(b + 1) * SQ, pl.ds(base, 2 * DH)] = (
                        jnp.concatenate(cs, axis=1))
                return carry

            lax.fori_loop(0, HPER // 2, head_pair, 0)
            acc[...] += lax.dot_general(
                ctx_ref[...], packs[t, :, HD:],
                (((1,), (1,)), ((), ())),
                preferred_element_type=jnp.float32)

        for k in (0, 1, 2):
            rdma(k).start()
        attend(0, my)

        wait_recv(1)
        rdma(3).start()
        wait_recv(2)
        rdma(4).start()
        wait_recv(3)
        attend(1, left)
        attend(2, right)
        attend(3, anti)

        wait_recv(4)
        rdma(5).start()
        wait_recv(5)
        rdma(6).start()
        attend(4, lax.rem(my + N_DEV - 2, N_DEV))
        attend(5, lax.rem(my + 2, N_DEV))

        wait_recv(6)
        wait_recv(7)
        attend(6, lax.rem(my + N_DEV - 3, N_DEV))
        attend(7, lax.rem(my + 3, N_DEV))

        for k in range(7):
            rdma(k).wait_send()

        for b in range(B):
            out_ref[b] = acc[b * SQ:(b + 1) * SQ, :].astype(jnp.bfloat16)

    return pl.pallas_call(
        body,
        out_shape=jax.ShapeDtypeStruct((B, SQ, DM), jnp.bfloat16),
        in_specs=[pl.BlockSpec(memory_space=pltpu.VMEM)] * 4,
        out_specs=pl.BlockSpec(memory_space=pltpu.VMEM),
        scratch_shapes=[
            pltpu.VMEM((N_DEV, DM, PACKW), jnp.bfloat16),
            pltpu.VMEM((ROWS, HD), jnp.bfloat16),
            pltpu.VMEM((ROWS, HD), jnp.bfloat16),
            pltpu.VMEM((ROWS, DM), jnp.float32),
            pltpu.SemaphoreType.DMA((7,)),
            pltpu.SemaphoreType.DMA((N_DEV,)),
        ],
        compiler_params=_CompilerParams(
            collective_id=0, vmem_limit_bytes=56 * 1024 * 1024),
    )(xb, pk, kt, vt)


# device time: 120394 ns/iter; 1.2442x vs baseline; 1.2442x over previous
import jax
import jax.numpy as jnp
from jax import lax
from jax.experimental import pallas as pl
from jax.experimental.pallas import tpu as pltpu

N_DEV = 8

_sem_signal = getattr(pl, "semaphore_signal", None) or pltpu.semaphore_signal
_sem_wait = getattr(pl, "semaphore_wait", None) or pltpu.semaphore_wait
_DID = getattr(pl, "DeviceIdType", None) or pltpu.DeviceIdType
_CompilerParams = getattr(pltpu, "CompilerParams", None) or pltpu.TPUCompilerParams


def kernel(x, Wq, K_ext, V_ext, Wo):
    B, SQ, DM = x.shape
    _, HD = Wq.shape
    _, SKV, H, DH = K_ext.shape
    HPER = HD // DH
    HALF = HD // 2
    ROWS = B * SQ

    xb = x.astype(jnp.bfloat16).reshape(ROWS, DM)
    wqb = Wq.astype(jnp.bfloat16) * jnp.bfloat16(0.125)
    wot = Wo.astype(jnp.bfloat16).T
    packA = jnp.concatenate([wqb[:, :HALF], wot[:, :HALF]], axis=1)
    packB = jnp.concatenate([wqb[:, HALF:], wot[:, HALF:]], axis=1)
    kt = jnp.transpose(K_ext, (0, 2, 3, 1)).astype(jnp.bfloat16)
    vt = jnp.transpose(V_ext, (0, 2, 3, 1)).astype(jnp.bfloat16)

    def body(x_ref, pa_ref, pb_ref, k_ref, v_ref, out_ref,
             cA, cB, ctx_ref, acc, sA, rA, sB, rB):
        my = lax.axis_index("i")
        left = lax.rem(my + N_DEV - 1, N_DEV)
        right = lax.rem(my + 1, N_DEV)

        barrier_sem = pltpu.get_barrier_semaphore()
        for nbr in (left, right):
            _sem_signal(barrier_sem, inc=1, device_id=(nbr,),
                        device_id_type=_DID.MESH)
        _sem_wait(barrier_sem, 2)

        cA[0] = pa_ref[...]
        cB[0] = pb_ref[...]
        acc[...] = jnp.zeros_like(acc)

        r4 = (lax.broadcasted_iota(jnp.int32, (SQ, SKV), 0) // 64) % 4
        c4 = (lax.broadcasted_iota(jnp.int32, (SQ, SKV), 1) // 64) % 4
        bias = jnp.where(r4 == c4, 0.0, -1e9).astype(jnp.float32)

        def make_rdmas(slot, nslot):
            a = pltpu.make_async_remote_copy(
                src_ref=cA.at[slot], dst_ref=cA.at[nslot],
                send_sem=sA.at[slot], recv_sem=rA.at[nslot],
                device_id=(right,), device_id_type=_DID.MESH)
            b = pltpu.make_async_remote_copy(
                src_ref=cB.at[slot], dst_ref=cB.at[nslot],
                send_sem=sB.at[slot], recv_sem=rB.at[nslot],
                device_id=(left,), device_id_type=_DID.MESH)
            return a, b

        def attend(comm, head_base, col0):
            q = jnp.dot(x_ref[...], comm[:, :HALF],
                        preferred_element_type=jnp.float32)
            q = q.astype(jnp.bfloat16)
            for b in range(B):
                for hl in range(HPER // 2):
                    khead = head_base + hl
                    qbh = q[b * SQ:(b + 1) * SQ, hl * DH:(hl + 1) * DH]
                    kbh = k_ref[b, khead]
                    s = lax.dot_general(
                        qbh, kbh, (((1,), (0,)), ((), ())),
                        preferred_element_type=jnp.float32)
                    e = jnp.exp(s + bias)
                    den = jnp.sum(e, axis=1, keepdims=True)
                    vbh = v_ref[b, khead]
                    c = lax.dot_general(
                        e.astype(jnp.bfloat16), vbh,
                        (((1,), (1,)), ((), ())),
                        preferred_element_type=jnp.float32)
                    c = c * (1.0 / den)
                    ctx_ref[b * SQ:(b + 1) * SQ,
                            col0 + hl * DH:col0 + (hl + 1) * DH] = (
                        c.astype(jnp.bfloat16))
            acc[...] += lax.dot_general(
                ctx_ref[:, col0:col0 + HALF], comm[:, HALF:],
                (((1,), (1,)), ((), ())),
                preferred_element_type=jnp.float32)

        def hop(h, carry):
            slot = lax.rem(h, 2)
            nslot = 1 - slot
            jA = lax.rem(my - h + N_DEV, N_DEV)
            jB = lax.rem(my + h, N_DEV)

            @pl.when(h < N_DEV - 1)
            def _send():
                for r in make_rdmas(slot, nslot):
                    r.start()

            attend(cA[slot], jA * HPER, 0)
            attend(cB[slot], jB * HPER + HPER // 2, HALF)

            @pl.when(h < N_DEV - 1)
            def _wait():
                for r in make_rdmas(slot, nslot):
                    r.wait()

            return carry

        lax.fori_loop(0, N_DEV, hop, 0)

        for b in range(B):
            out_ref[b] = acc[b * SQ:(b + 1) * SQ, :].astype(jnp.bfloat16)

    return pl.pallas_call(
        body,
        out_shape=jax.ShapeDtypeStruct((B, SQ, DM), jnp.bfloat16),
        in_specs=[pl.BlockSpec(memory_space=pltpu.VMEM)] * 5,
        out_specs=pl.BlockSpec(memory_space=pltpu.VMEM),
        scratch_shapes=[
            pltpu.VMEM((2, DM, HD), jnp.bfloat16),
            pltpu.VMEM((2, DM, HD), jnp.bfloat16),
            pltpu.VMEM((ROWS, HD), jnp.bfloat16),
            pltpu.VMEM((ROWS, DM), jnp.float32),
            pltpu.SemaphoreType.DMA((2,)),
            pltpu.SemaphoreType.DMA((2,)),
            pltpu.SemaphoreType.DMA((2,)),
            pltpu.SemaphoreType.DMA((2,)),
        ],
        compiler_params=_CompilerParams(
            collective_id=0, vmem_limit_bytes=56 * 1024 * 1024),
    )(xb, packA, packB, kt, vt)
